# Initial kernel scaffold; baseline (speedup 1.0000x reference)
import functools

import jax
import jax.numpy as jnp
from jax import lax
from jax.experimental import pallas as pl
from jax.experimental.pallas import tpu as pltpu
from jax.experimental.pallas import tpu_sc as plsc

NN = 10000
NE = 320000
NL = 4
HSTEP = 0.1
EPS = 1e-5
NC, NS = 2, 16
NW = NC * NS
GCH = NE // 128
SCH = 2 * NE // 128
NPT = NN // NS

_SC_MESH = dict(core_axis_name="c", subcore_axis_name="s",
                num_cores=NC, num_subcores=NS)


def _sc_gather(yi, yj, ii2d, jj2d):
    @functools.partial(
        pl.kernel,
        out_type=(jax.ShapeDtypeStruct((NE, 32), jnp.float32),
                  jax.ShapeDtypeStruct((NE, 32), jnp.float32)),
        mesh=plsc.VectorSubcoreMesh(**_SC_MESH),
        scratch_types=[
            pltpu.VMEM((2, 128), jnp.int32),
            pltpu.VMEM((128, 32), jnp.float32),
            pltpu.VMEM((128, 32), jnp.float32),
            pltpu.SemaphoreType.DMA,
            pltpu.SemaphoreType.DMA,
        ],
    )
    def k(yi_h, yj_h, ii_h, jj_h, ti_h, tj_h, idx_v, bi_v, bj_v, s1, s2):
        w = lax.axis_index("s") * NC + lax.axis_index("c")
        nw = 78 + jnp.where(w < GCH - 78 * NW, 1, 0)

        def body(i, carry):
            c = w + i * NW
            pltpu.sync_copy(ii_h.at[c], idx_v.at[0])
            pltpu.sync_copy(jj_h.at[c], idx_v.at[1])
            cp1 = pltpu.async_copy(yi_h.at[idx_v.at[0]], bi_v, s1)
            cp2 = pltpu.async_copy(yj_h.at[idx_v.at[1]], bj_v, s2)
            cp1.wait()
            cp2.wait()
            pltpu.sync_copy(bi_v, ti_h.at[pl.ds(c * 128, 128)])
            pltpu.sync_copy(bj_v, tj_h.at[pl.ds(c * 128, 128)])
            return carry

        lax.fori_loop(0, nw, body, 0)

    return k(yi, yj, ii2d, jj2d)


def _sc_scatter(z2, sidx2d, zrows):
    @functools.partial(
        pl.kernel,
        out_type=jax.ShapeDtypeStruct((NC, NN, 32), jnp.float32),
        mesh=plsc.VectorSubcoreMesh(**_SC_MESH),
        scratch_types=[
            pltpu.VMEM((2, 128), jnp.int32),
            pltpu.VMEM((128, 32), jnp.float32),
            pltpu.VMEM_SHARED((NN, 32), jnp.float32),
        ],
    )
    def k(z_h, sidx_h, zero_h, out_h, idx_v, val_v, acc_sh):
        cid = lax.axis_index("c")
        sid = lax.axis_index("s")
        w = sid * NC + cid
        sl = pl.ds(sid * NPT, NPT)
        pltpu.sync_copy(zero_h.at[sl], acc_sh.at[sl])
        plsc.subcore_barrier()
        nw = 156 + jnp.where(w < SCH - 156 * NW, 1, 0)

        def body(i, carry):
            c = w + i * NW
            pltpu.sync_copy(sidx_h.at[c], idx_v.at[0])
            pltpu.sync_copy(z_h.at[pl.ds(c * 128, 128)], val_v)
            pltpu.sync_copy(val_v, acc_sh.at[idx_v.at[0]], add=True)
            return carry

        lax.fori_loop(0, nw, body, 0)
        plsc.subcore_barrier()
        pltpu.sync_copy(acc_sh.at[sl], out_h.at[cid, sl])

    return k(z2, sidx2d, zrows)


def _open_body(xnr, wo, wi, wj, xn0, yi, yj):
    x = jnp.dot(xnr[...], wo[...], preferred_element_type=jnp.float32)
    xn0[...] = x
    yi[...] = jnp.dot(x, wi[...], preferred_element_type=jnp.float32)
    yj[...] = jnp.dot(x, wj[...], preferred_element_type=jnp.float32)


def _tc_open(xn_rows, wopenT, wiT, wjT):
    return pl.pallas_call(
        _open_body,
        out_shape=(jax.ShapeDtypeStruct((NN, 128), jnp.float32),
                   jax.ShapeDtypeStruct((NN, 32), jnp.float32),
                   jax.ShapeDtypeStruct((NN, 32), jnp.float32)),
    )(xn_rows, wopenT, wiT, wjT)


_SBLK = 8000
_SGRID = NE // _SBLK


def _stats_body(ti, tj, t, s, q):
    x = ti[...] + tj[...]
    t[...] = x
    s[...] = jnp.full((1, 1, 128), jnp.sum(x), jnp.float32)
    q[...] = jnp.full((1, 1, 128), jnp.sum(x * x), jnp.float32)


def _tc_add_stats(ti, tj):
    t, s, q = pl.pallas_call(
        _stats_body,
        grid=(_SGRID,),
        in_specs=[pl.BlockSpec((_SBLK, 32), lambda i: (i, 0)),
                  pl.BlockSpec((_SBLK, 32), lambda i: (i, 0))],
        out_specs=(pl.BlockSpec((_SBLK, 32), lambda i: (i, 0)),
                   pl.BlockSpec((1, 1, 128), lambda i: (i, 0, 0)),
                   pl.BlockSpec((1, 1, 128), lambda i: (i, 0, 0))),
        out_shape=(jax.ShapeDtypeStruct((NE, 32), jnp.float32),
                   jax.ShapeDtypeStruct((_SGRID, 1, 128), jnp.float32),
                   jax.ShapeDtypeStruct((_SGRID, 1, 128), jnp.float32)),
    )(ti, tj)
    ssum = jnp.sum(s[:, 0, 0])
    qsum = jnp.sum(q[:, 0, 0])
    m = ssum / (NE * 32)
    v = qsum / (NE * 32) - m * m
    a = lax.rsqrt(v + EPS)
    return t, a, -m * a


_ZBLK = 4000
_ZGRID = NE // _ZBLK


def _z_body(nk, final, *refs):
    scal, xe0 = refs[0], refs[1]
    ts = refs[2:2 + nk]
    if final:
        p, qs, p2, q2s, zout, xeout = refs[2 + nk:]
    else:
        p, qs, zout = refs[2 + nk:]
    acc = jnp.dot(xe0[...], p[...], preferred_element_type=jnp.float32)
    if final:
        acc2 = jnp.dot(xe0[...], p2[...], preferred_element_type=jnp.float32)
    for k in range(nk):
        a = scal[k, 0]
        b = scal[k, 1]
        r = jnp.maximum(ts[k][...] * a + b, 0.0)
        acc = acc + jnp.dot(r, qs[k], preferred_element_type=jnp.float32)
        if final:
            acc2 = acc2 + jnp.dot(r, q2s[k], preferred_element_type=jnp.float32)
    zout[...] = acc
    if final:
        xeout[...] = acc2


def _tc_zmat(scal, xe0r, ts, p, qs, p2=None, q2s=None):
    nk = len(ts)
    final = p2 is not None
    in_specs = [pl.BlockSpec(memory_space=pltpu.SMEM),
                pl.BlockSpec((_ZBLK, 16), lambda i: (i, 0))]
    in_specs += [pl.BlockSpec((_ZBLK, 32), lambda i: (i, 0))] * nk
    in_specs += [pl.BlockSpec((16, 64), lambda i: (0, 0)),
                 pl.BlockSpec((nk, 32, 64), lambda i: (0, 0, 0))]
    args = [scal, xe0r] + list(ts) + [p, qs]
    out_specs = [pl.BlockSpec((_ZBLK, 64), lambda i: (i, 0))]
    out_shape = [jax.ShapeDtypeStruct((NE, 64), jnp.float32)]
    if final:
        in_specs += [pl.BlockSpec((16, 16), lambda i: (0, 0)),
                     pl.BlockSpec((nk, 32, 16), lambda i: (0, 0, 0))]
        args += [p2, q2s]
        out_specs += [pl.BlockSpec((_ZBLK, 16), lambda i: (i, 0))]
        out_shape += [jax.ShapeDtypeStruct((NE, 16), jnp.float32)]
    out = pl.pallas_call(
        functools.partial(_z_body, nk, final),
        grid=(_ZGRID,),
        in_specs=in_specs,
        out_specs=tuple(out_specs) if final else out_specs[0],
        out_shape=tuple(out_shape) if final else out_shape[0],
    )(*args)
    return out if final else (out, None)


def _node_body(final, accp, xn, kn2T, wa, wb, *outs):
    u = accp[0] + accp[1]
    m = jnp.mean(u)
    v = jnp.mean(u * u) - m * m
    r = jnp.maximum((u - m) * lax.rsqrt(v + EPS), 0.0)
    x = xn[...] + HSTEP * jnp.dot(r, kn2T[...], preferred_element_type=jnp.float32)
    if final:
        outs[0][...] = jnp.dot(x, wa[...], preferred_element_type=jnp.float32)
    else:
        outs[0][...] = x
        outs[1][...] = jnp.dot(x, wa[...], preferred_element_type=jnp.float32)
        outs[2][...] = jnp.dot(x, wb[...], preferred_element_type=jnp.float32)


def _tc_node_update(accp, xn, kn2T, wa, wb, final):
    if final:
        out_shape = jax.ShapeDtypeStruct((NN, 128), jnp.float32)
    else:
        out_shape = (jax.ShapeDtypeStruct((NN, 128), jnp.float32),
                     jax.ShapeDtypeStruct((NN, 32), jnp.float32),
                     jax.ShapeDtypeStruct((NN, 32), jnp.float32))
    return pl.pallas_call(
        functools.partial(_node_body, final),
        out_shape=out_shape,
    )(accp, xn, kn2T, wa, wb)


def kernel(xn, xe, edge_index, KNopen, KEopen, KE1, KE2, KN1, KN2,
           KNclose, KEclose):
    f32 = jnp.float32
    iInd = edge_index[0].astype(jnp.int32)
    jInd = edge_index[1].astype(jnp.int32)
    ii2d = iInd.reshape(GCH, 128)
    jj2d = jInd.reshape(GCH, 128)
    sidx2d = jnp.stack([iInd, jInd], axis=1).reshape(SCH, 128)
    zrows = jnp.zeros((NN, 32), f32)

    xn_rows = jnp.transpose(xn[0]).astype(f32)
    xe_rows = jnp.transpose(xe[0]).astype(f32)

    wiT, wjT, VT = [], [], []
    for l in range(NL):
        a, b = KE1[l][:, :128], KE1[l][:, 128:]
        wiT.append(jnp.transpose(a + 0.5 * b))
        wjT.append(jnp.transpose(-a + 0.5 * b))
        na, nb = KN1[l][:, :128], KN1[l][:, 128:]
        VT.append(jnp.concatenate([jnp.transpose(na + 0.5 * nb),
                                   jnp.transpose(-na + 0.5 * nb)], axis=1))
    P = [jnp.transpose(KEopen) @ VT[l] for l in range(NL)]
    Q = [jnp.stack([HSTEP * (jnp.transpose(KE2[k]) @ VT[l])
                    for k in range(l + 1)]) for l in range(NL)]
    P2 = jnp.transpose(KEopen) @ jnp.transpose(KEclose)
    Q2 = jnp.stack([HSTEP * (jnp.transpose(KE2[k]) @ jnp.transpose(KEclose))
                    for k in range(NL)])
    kn2T = [jnp.transpose(KN2[l]) for l in range(NL)]

    xnr, yi, yj = _tc_open(xn_rows, jnp.transpose(KNopen), wiT[0], wjT[0])

    ts, scals = [], []
    xe_out_rows = None
    xn_out_rows = None
    for l in range(NL):
        ti, tj = _sc_gather(yi, yj, ii2d, jj2d)
        t, a, b = _tc_add_stats(ti, tj)
        ts.append(t)
        scals.append(jnp.stack([a, b]))
        scal = jnp.stack(scals)
        final = l == NL - 1
        z, xe_out_rows_maybe = _tc_zmat(
            scal, xe_rows, ts, P[l], Q[l],
            P2 if final else None, Q2 if final else None)
        if final:
            xe_out_rows = xe_out_rows_maybe
        accp = _sc_scatter(z.reshape(2 * NE, 32), sidx2d, zrows)
        if final:
            xn_out_rows = _tc_node_update(
                accp, xnr, kn2T[l], jnp.transpose(KNclose),
                jnp.transpose(KNclose), final=True)
        else:
            xnr, yi, yj = _tc_node_update(
                accp, xnr, kn2T[l], wiT[l + 1], wjT[l + 1], final=False)

    xn_out = jnp.transpose(xn_out_rows)[None]
    xe_out = jnp.transpose(xe_out_rows)[None]
    return (xn_out, xe_out)


# speedup vs baseline: 3.2208x; 3.2208x over previous
import functools

import jax
import jax.numpy as jnp
from jax import lax
from jax.experimental import pallas as pl
from jax.experimental.pallas import tpu as pltpu
from jax.experimental.pallas import tpu_sc as plsc

NN = 10000
NE = 320000
NL = 4
HSTEP = 0.1
EPS = 1e-5
NC, NS = 2, 16
NW = NC * NS
GCH = NE // 128
SCH = 2 * NE // 128
NPT = NN // NS

_SC_MESH = dict(core_axis_name="c", subcore_axis_name="s",
                num_cores=NC, num_subcores=NS)
_SC_PARAMS = pltpu.CompilerParams(use_tc_tiling_on_sc=False)


def _sc_gather(yi, yj, ii2d, jj2d):
    @functools.partial(
        pl.kernel,
        out_type=(jax.ShapeDtypeStruct((NE, 32), jnp.float32),
                  jax.ShapeDtypeStruct((NE, 32), jnp.float32)),
        mesh=plsc.VectorSubcoreMesh(**_SC_MESH),
        scratch_types=[
            pltpu.VMEM((2, 128), jnp.int32),
            pltpu.VMEM((128, 32), jnp.float32),
            pltpu.VMEM((128, 32), jnp.float32),
            pltpu.SemaphoreType.DMA,
            pltpu.SemaphoreType.DMA,
        ],
        compiler_params=_SC_PARAMS,
    )
    def k(yi_h, yj_h, ii_h, jj_h, ti_h, tj_h, idx_v, bi_v, bj_v, s1, s2):
        w = lax.axis_index("s") * NC + lax.axis_index("c")
        nw = 78 + jnp.where(w < GCH - 78 * NW, 1, 0)

        def body(i, carry):
            c = w + i * NW
            pltpu.sync_copy(ii_h.at[c], idx_v.at[0])
            pltpu.sync_copy(jj_h.at[c], idx_v.at[1])
            cp1 = pltpu.async_copy(yi_h.at[idx_v.at[0]], bi_v, s1)
            cp2 = pltpu.async_copy(yj_h.at[idx_v.at[1]], bj_v, s2)
            cp1.wait()
            cp2.wait()
            pltpu.sync_copy(bi_v, ti_h.at[pl.ds(c * 128, 128)])
            pltpu.sync_copy(bj_v, tj_h.at[pl.ds(c * 128, 128)])
            return carry

        lax.fori_loop(0, nw, body, 0)

    return k(yi, yj, ii2d, jj2d)


def _sc_scatter(z2, sidx2d, zrows):
    @functools.partial(
        pl.kernel,
        out_type=jax.ShapeDtypeStruct((NC, NN, 32), jnp.float32),
        mesh=plsc.VectorSubcoreMesh(**_SC_MESH),
        scratch_types=[
            pltpu.VMEM((2, 128), jnp.int32),
            pltpu.VMEM((128, 32), jnp.float32),
            pltpu.VMEM_SHARED((NN, 32), jnp.float32),
        ],
        compiler_params=_SC_PARAMS,
    )
    def k(z_h, sidx_h, zero_h, out_h, idx_v, val_v, acc_sh):
        cid = lax.axis_index("c")
        sid = lax.axis_index("s")
        w = sid * NC + cid
        sl = pl.ds(sid * NPT, NPT)
        pltpu.sync_copy(zero_h.at[sl], acc_sh.at[sl])
        plsc.subcore_barrier()
        nw = 156 + jnp.where(w < SCH - 156 * NW, 1, 0)

        def body(i, carry):
            c = w + i * NW
            pltpu.sync_copy(sidx_h.at[c], idx_v.at[0])
            pltpu.sync_copy(z_h.at[pl.ds(c * 128, 128)], val_v)
            pltpu.sync_copy(val_v, acc_sh.at[idx_v.at[0]], add=True)
            return carry

        lax.fori_loop(0, nw, body, 0)
        plsc.subcore_barrier()
        pltpu.sync_copy(acc_sh.at[sl], out_h.at[cid, sl])

    return k(z2, sidx2d, zrows)


def _open_body(xnr, wo, wi, wj, xn0, yi, yj):
    x = jnp.dot(xnr[...], wo[...], preferred_element_type=jnp.float32)
    xn0[...] = x
    yi[...] = jnp.dot(x, wi[...], preferred_element_type=jnp.float32)
    yj[...] = jnp.dot(x, wj[...], preferred_element_type=jnp.float32)


def _tc_open(xn_rows, wopenT, wiT, wjT):
    return pl.pallas_call(
        _open_body,
        out_shape=(jax.ShapeDtypeStruct((NN, 128), jnp.float32),
                   jax.ShapeDtypeStruct((NN, 32), jnp.float32),
                   jax.ShapeDtypeStruct((NN, 32), jnp.float32)),
    )(xn_rows, wopenT, wiT, wjT)


_SBLK = 8000
_SGRID = NE // _SBLK


def _stats_body(ti, tj, t, s, q):
    x = ti[...] + tj[...]
    t[...] = x
    s[...] = jnp.full((1, 1, 128), jnp.sum(x), jnp.float32)
    q[...] = jnp.full((1, 1, 128), jnp.sum(x * x), jnp.float32)


def _tc_add_stats(ti, tj):
    t, s, q = pl.pallas_call(
        _stats_body,
        grid=(_SGRID,),
        in_specs=[pl.BlockSpec((_SBLK, 32), lambda i: (i, 0)),
                  pl.BlockSpec((_SBLK, 32), lambda i: (i, 0))],
        out_specs=(pl.BlockSpec((_SBLK, 32), lambda i: (i, 0)),
                   pl.BlockSpec((1, 1, 128), lambda i: (i, 0, 0)),
                   pl.BlockSpec((1, 1, 128), lambda i: (i, 0, 0))),
        out_shape=(jax.ShapeDtypeStruct((NE, 32), jnp.float32),
                   jax.ShapeDtypeStruct((_SGRID, 1, 128), jnp.float32),
                   jax.ShapeDtypeStruct((_SGRID, 1, 128), jnp.float32)),
    )(ti, tj)
    ssum = jnp.sum(s[:, 0, 0])
    qsum = jnp.sum(q[:, 0, 0])
    m = ssum / (NE * 32)
    v = qsum / (NE * 32) - m * m
    a = lax.rsqrt(v + EPS)
    return t, a, -m * a


_ZBLK = 4000
_ZGRID = NE // _ZBLK


def _z_body(nk, final, *refs):
    scal, xe0 = refs[0], refs[1]
    ts = refs[2:2 + nk]
    if final:
        p, qs, p2, q2s, zout, xeout = refs[2 + nk:]
    else:
        p, qs, zout = refs[2 + nk:]
    acc = jnp.dot(xe0[...], p[...], preferred_element_type=jnp.float32)
    if final:
        acc2 = jnp.dot(xe0[...], p2[...], preferred_element_type=jnp.float32)
    for k in range(nk):
        a = scal[k, 0]
        b = scal[k, 1]
        r = jnp.maximum(ts[k][...] * a + b, 0.0)
        acc = acc + jnp.dot(r, qs[k], preferred_element_type=jnp.float32)
        if final:
            acc2 = acc2 + jnp.dot(r, q2s[k], preferred_element_type=jnp.float32)
    zout[...] = acc
    if final:
        xeout[...] = acc2


def _tc_zmat(scal, xe0r, ts, p, qs, p2=None, q2s=None):
    nk = len(ts)
    final = p2 is not None
    in_specs = [pl.BlockSpec(memory_space=pltpu.SMEM),
                pl.BlockSpec((_ZBLK, 16), lambda i: (i, 0))]
    in_specs += [pl.BlockSpec((_ZBLK, 32), lambda i: (i, 0))] * nk
    in_specs += [pl.BlockSpec((16, 64), lambda i: (0, 0)),
                 pl.BlockSpec((nk, 32, 64), lambda i: (0, 0, 0))]
    args = [scal, xe0r] + list(ts) + [p, qs]
    out_specs = [pl.BlockSpec((_ZBLK, 64), lambda i: (i, 0))]
    out_shape = [jax.ShapeDtypeStruct((NE, 64), jnp.float32)]
    if final:
        in_specs += [pl.BlockSpec((16, 16), lambda i: (0, 0)),
                     pl.BlockSpec((nk, 32, 16), lambda i: (0, 0, 0))]
        args += [p2, q2s]
        out_specs += [pl.BlockSpec((_ZBLK, 16), lambda i: (i, 0))]
        out_shape += [jax.ShapeDtypeStruct((NE, 16), jnp.float32)]
    out = pl.pallas_call(
        functools.partial(_z_body, nk, final),
        grid=(_ZGRID,),
        in_specs=in_specs,
        out_specs=tuple(out_specs) if final else out_specs[0],
        out_shape=tuple(out_shape) if final else out_shape[0],
    )(*args)
    return out if final else (out, None)


def _node_body(final, accp, xn, kn2T, wa, wb, *outs):
    u = accp[0] + accp[1]
    m = jnp.mean(u)
    v = jnp.mean(u * u) - m * m
    r = jnp.maximum((u - m) * lax.rsqrt(v + EPS), 0.0)
    x = xn[...] + HSTEP * jnp.dot(r, kn2T[...], preferred_element_type=jnp.float32)
    if final:
        outs[0][...] = jnp.dot(x, wa[...], preferred_element_type=jnp.float32)
    else:
        outs[0][...] = x
        outs[1][...] = jnp.dot(x, wa[...], preferred_element_type=jnp.float32)
        outs[2][...] = jnp.dot(x, wb[...], preferred_element_type=jnp.float32)


def _tc_node_update(accp, xn, kn2T, wa, wb, final):
    if final:
        out_shape = jax.ShapeDtypeStruct((NN, 128), jnp.float32)
    else:
        out_shape = (jax.ShapeDtypeStruct((NN, 128), jnp.float32),
                     jax.ShapeDtypeStruct((NN, 32), jnp.float32),
                     jax.ShapeDtypeStruct((NN, 32), jnp.float32))
    return pl.pallas_call(
        functools.partial(_node_body, final),
        out_shape=out_shape,
    )(accp, xn, kn2T, wa, wb)


def kernel(xn, xe, edge_index, KNopen, KEopen, KE1, KE2, KN1, KN2,
           KNclose, KEclose):
    f32 = jnp.float32
    iInd = edge_index[0].astype(jnp.int32)
    jInd = edge_index[1].astype(jnp.int32)
    ii2d = iInd.reshape(GCH, 128)
    jj2d = jInd.reshape(GCH, 128)
    sidx2d = jnp.stack([iInd, jInd], axis=1).reshape(SCH, 128)
    zrows = jnp.zeros((NN, 32), f32)

    xn_rows = jnp.transpose(xn[0]).astype(f32)
    xe_rows = jnp.transpose(xe[0]).astype(f32)

    wiT, wjT, VT = [], [], []
    for l in range(NL):
        a, b = KE1[l][:, :128], KE1[l][:, 128:]
        wiT.append(jnp.transpose(a + 0.5 * b))
        wjT.append(jnp.transpose(-a + 0.5 * b))
        na, nb = KN1[l][:, :128], KN1[l][:, 128:]
        VT.append(jnp.concatenate([jnp.transpose(na + 0.5 * nb),
                                   jnp.transpose(-na + 0.5 * nb)], axis=1))
    P = [jnp.transpose(KEopen) @ VT[l] for l in range(NL)]
    Q = [jnp.stack([HSTEP * (jnp.transpose(KE2[k]) @ VT[l])
                    for k in range(l + 1)]) for l in range(NL)]
    P2 = jnp.transpose(KEopen) @ jnp.transpose(KEclose)
    Q2 = jnp.stack([HSTEP * (jnp.transpose(KE2[k]) @ jnp.transpose(KEclose))
                    for k in range(NL)])
    kn2T = [jnp.transpose(KN2[l]) for l in range(NL)]

    xnr, yi, yj = _tc_open(xn_rows, jnp.transpose(KNopen), wiT[0], wjT[0])

    ts, scals = [], []
    xe_out_rows = None
    xn_out_rows = None
    for l in range(NL):
        ti, tj = _sc_gather(yi, yj, ii2d, jj2d)
        t, a, b = _tc_add_stats(ti, tj)
        ts.append(t)
        scals.append(jnp.stack([a, b]))
        scal = jnp.stack(scals)
        final = l == NL - 1
        z, xe_out_rows_maybe = _tc_zmat(
            scal, xe_rows, ts, P[l], Q[l],
            P2 if final else None, Q2 if final else None)
        if final:
            xe_out_rows = xe_out_rows_maybe
        accp = _sc_scatter(z.reshape(2 * NE, 32), sidx2d, zrows)
        if final:
            xn_out_rows = _tc_node_update(
                accp, xnr, kn2T[l], jnp.transpose(KNclose),
                jnp.transpose(KNclose), final=True)
        else:
            xnr, yi, yj = _tc_node_update(
                accp, xnr, kn2T[l], wiT[l + 1], wjT[l + 1], final=False)

    xn_out = jnp.transpose(xn_out_rows)[None]
    xe_out = jnp.transpose(xe_out_rows)[None]
    return (xn_out, xe_out)


# speedup vs baseline: 4.1283x; 1.2818x over previous
import functools

import jax
import jax.numpy as jnp
from jax import lax
from jax.experimental import pallas as pl
from jax.experimental.pallas import tpu as pltpu
from jax.experimental.pallas import tpu_sc as plsc

NN = 10000
NE = 320000
NL = 4
HSTEP = 0.1
EPS = 1e-5
NC, NS = 2, 16
NW = NC * NS
GCH = NE // 128
SCH = 2 * NE // 128
NPT = NN // NS

_SC_MESH = dict(core_axis_name="c", subcore_axis_name="s",
                num_cores=NC, num_subcores=NS)
_SC_PARAMS = pltpu.CompilerParams(use_tc_tiling_on_sc=False)


_GK = 4
_GG = GCH // _GK
_GBASE = _GG // NW
_GEXTRA = _GG - _GBASE * NW
_GMAXCH = (_GBASE + 1) * _GK


def _sc_gather(yi, yj, ii2d, jj2d):
    @functools.partial(
        pl.kernel,
        out_type=(jax.ShapeDtypeStruct((NE, 32), jnp.float32),
                  jax.ShapeDtypeStruct((NE, 32), jnp.float32)),
        mesh=plsc.VectorSubcoreMesh(**_SC_MESH),
        scratch_types=[
            pltpu.VMEM((_GMAXCH, 128), jnp.int32),
            pltpu.VMEM((_GMAXCH, 128), jnp.int32),
            pltpu.VMEM((2, _GK, 128, 32), jnp.float32),
            pltpu.VMEM((2, _GK, 128, 32), jnp.float32),
            pltpu.SemaphoreType.DMA,
            pltpu.SemaphoreType.DMA,
            pltpu.SemaphoreType.DMA,
            pltpu.SemaphoreType.DMA,
        ],
        compiler_params=_SC_PARAMS,
    )
    def k(yi_h, yj_h, ii_h, jj_h, ti_h, tj_h,
          idxi_v, idxj_v, bi_v, bj_v, sg0, sg1, sw0, sw1):
        w = lax.axis_index("s") * NC + lax.axis_index("c")
        ng = _GBASE + jnp.where(w < _GEXTRA, 1, 0)
        c0 = (_GBASE * w + jnp.minimum(w, _GEXTRA)) * _GK

        nbase = _GBASE * _GK
        pltpu.sync_copy(ii_h.at[pl.ds(c0, nbase)], idxi_v.at[pl.ds(0, nbase)])
        pltpu.sync_copy(jj_h.at[pl.ds(c0, nbase)], idxj_v.at[pl.ds(0, nbase)])

        @pl.when(w < _GEXTRA)
        def _():
            pltpu.sync_copy(ii_h.at[pl.ds(c0 + nbase, _GK)],
                            idxi_v.at[pl.ds(nbase, _GK)])
            pltpu.sync_copy(jj_h.at[pl.ds(c0 + nbase, _GK)],
                            idxj_v.at[pl.ds(nbase, _GK)])

        sg = (sg0, sg1)
        sw = (sw0, sw1)

        def issue_g(g, bank):
            for b in range(_GK):
                ch = g * _GK + b
                pltpu.async_copy(yi_h.at[idxi_v.at[ch]], bi_v.at[bank, b], sg[bank])
                pltpu.async_copy(yj_h.at[idxj_v.at[ch]], bj_v.at[bank, b], sg[bank])

        def wait_g(bank):
            for b in range(_GK):
                pltpu.make_async_copy(yi_h.at[idxi_v.at[0]], bi_v.at[bank, b], sg[bank]).wait()
                pltpu.make_async_copy(yj_h.at[idxj_v.at[0]], bj_v.at[bank, b], sg[bank]).wait()

        def issue_w(g, bank):
            for b in range(_GK):
                row = (c0 + g * _GK + b) * 128
                pltpu.async_copy(bi_v.at[bank, b], ti_h.at[pl.ds(row, 128)], sw[bank])
                pltpu.async_copy(bj_v.at[bank, b], tj_h.at[pl.ds(row, 128)], sw[bank])

        def wait_w(bank):
            for b in range(_GK):
                pltpu.make_async_copy(bi_v.at[bank, b], ti_h.at[pl.ds(0, 128)], sw[bank]).wait()
                pltpu.make_async_copy(bj_v.at[bank, b], tj_h.at[pl.ds(0, 128)], sw[bank]).wait()

        issue_g(0, 0)

        def pair(p, carry):
            g0 = 2 * p
            g1 = 2 * p + 1
            g2 = 2 * p + 2

            @pl.when(g1 < ng)
            def _():
                @pl.when(p > 0)
                def _():
                    wait_w(1)
                issue_g(g1, 1)

            wait_g(0)
            issue_w(g0, 0)

            @pl.when(g1 < ng)
            def _():
                @pl.when(g2 < ng)
                def _():
                    wait_w(0)
                    issue_g(g2, 0)
                wait_g(1)
                issue_w(g1, 1)

            return carry

        lax.fori_loop(0, (_GBASE + 1) // 2, pair, 0)
        wait_w(0)
        wait_w(1)

    return k(yi, yj, ii2d, jj2d)


_SG = SCH // _GK
_SBASE = _SG // NW
_SEXTRA = _SG - _SBASE * NW
_SMAXCH = (_SBASE + 1) * _GK


def _sc_scatter(z2, sidx2d, zrows):
    @functools.partial(
        pl.kernel,
        out_type=jax.ShapeDtypeStruct((NC, NN, 32), jnp.float32),
        mesh=plsc.VectorSubcoreMesh(**_SC_MESH),
        scratch_types=[
            pltpu.VMEM((_SMAXCH, 128), jnp.int32),
            pltpu.VMEM((2, _GK, 128, 32), jnp.float32),
            pltpu.VMEM_SHARED((NN, 32), jnp.float32),
            pltpu.SemaphoreType.DMA,
            pltpu.SemaphoreType.DMA,
            pltpu.SemaphoreType.DMA,
            pltpu.SemaphoreType.DMA,
        ],
        compiler_params=_SC_PARAMS,
    )
    def k(z_h, sidx_h, zero_h, out_h, idx_v, val_v, acc_sh,
          sv0, sv1, ss0, ss1):
        cid = lax.axis_index("c")
        sid = lax.axis_index("s")
        w = sid * NC + cid
        sl = pl.ds(sid * NPT, NPT)
        pltpu.sync_copy(zero_h.at[sl], acc_sh.at[sl])
        ng = _SBASE + jnp.where(w < _SEXTRA, 1, 0)
        c0 = (_SBASE * w + jnp.minimum(w, _SEXTRA)) * _GK

        nbase = _SBASE * _GK
        pltpu.sync_copy(sidx_h.at[pl.ds(c0, nbase)], idx_v.at[pl.ds(0, nbase)])

        @pl.when(w < _SEXTRA)
        def _():
            pltpu.sync_copy(sidx_h.at[pl.ds(c0 + nbase, _GK)],
                            idx_v.at[pl.ds(nbase, _GK)])

        plsc.subcore_barrier()

        sv = (sv0, sv1)
        ss = (ss0, ss1)

        def issue_v(g, bank):
            for b in range(_GK):
                row = (c0 + g * _GK + b) * 128
                pltpu.async_copy(z_h.at[pl.ds(row, 128)], val_v.at[bank, b], sv[bank])

        def wait_v(bank):
            for b in range(_GK):
                pltpu.make_async_copy(z_h.at[pl.ds(0, 128)], val_v.at[bank, b], sv[bank]).wait()

        def issue_s(g, bank):
            for b in range(_GK):
                ch = g * _GK + b
                pltpu.async_copy(val_v.at[bank, b], acc_sh.at[idx_v.at[ch]],
                                 ss[bank], add=True)

        def wait_s(bank):
            for b in range(_GK):
                pltpu.make_async_copy(val_v.at[bank, b], acc_sh.at[idx_v.at[0]],
                                      ss[bank]).wait()

        issue_v(0, 0)

        def pair(p, carry):
            g0 = 2 * p
            g1 = 2 * p + 1
            g2 = 2 * p + 2

            @pl.when(g1 < ng)
            def _():
                @pl.when(p > 0)
                def _():
                    wait_s(1)
                issue_v(g1, 1)

            wait_v(0)
            issue_s(g0, 0)

            @pl.when(g1 < ng)
            def _():
                @pl.when(g2 < ng)
                def _():
                    wait_s(0)
                    issue_v(g2, 0)
                wait_v(1)
                issue_s(g1, 1)

            return carry

        lax.fori_loop(0, (_SBASE + 1) // 2, pair, 0)
        wait_s(0)
        wait_s(1)
        plsc.subcore_barrier()
        pltpu.sync_copy(acc_sh.at[sl], out_h.at[cid, sl])

    return k(z2, sidx2d, zrows)


def _open_body(xnr, wo, wi, wj, xn0, yi, yj):
    x = jnp.dot(xnr[...], wo[...], preferred_element_type=jnp.float32)
    xn0[...] = x
    yi[...] = jnp.dot(x, wi[...], preferred_element_type=jnp.float32)
    yj[...] = jnp.dot(x, wj[...], preferred_element_type=jnp.float32)


def _tc_open(xn_rows, wopenT, wiT, wjT):
    return pl.pallas_call(
        _open_body,
        out_shape=(jax.ShapeDtypeStruct((NN, 128), jnp.float32),
                   jax.ShapeDtypeStruct((NN, 32), jnp.float32),
                   jax.ShapeDtypeStruct((NN, 32), jnp.float32)),
    )(xn_rows, wopenT, wiT, wjT)


_SBLK = 8000
_SGRID = NE // _SBLK


def _stats_body(ti, tj, t, s, q):
    x = ti[...] + tj[...]
    t[...] = x
    s[...] = jnp.full((1, 1, 128), jnp.sum(x), jnp.float32)
    q[...] = jnp.full((1, 1, 128), jnp.sum(x * x), jnp.float32)


def _tc_add_stats(ti, tj):
    t, s, q = pl.pallas_call(
        _stats_body,
        grid=(_SGRID,),
        in_specs=[pl.BlockSpec((_SBLK, 32), lambda i: (i, 0)),
                  pl.BlockSpec((_SBLK, 32), lambda i: (i, 0))],
        out_specs=(pl.BlockSpec((_SBLK, 32), lambda i: (i, 0)),
                   pl.BlockSpec((1, 1, 128), lambda i: (i, 0, 0)),
                   pl.BlockSpec((1, 1, 128), lambda i: (i, 0, 0))),
        out_shape=(jax.ShapeDtypeStruct((NE, 32), jnp.float32),
                   jax.ShapeDtypeStruct((_SGRID, 1, 128), jnp.float32),
                   jax.ShapeDtypeStruct((_SGRID, 1, 128), jnp.float32)),
    )(ti, tj)
    ssum = jnp.sum(s[:, 0, 0])
    qsum = jnp.sum(q[:, 0, 0])
    m = ssum / (NE * 32)
    v = qsum / (NE * 32) - m * m
    a = lax.rsqrt(v + EPS)
    return t, a, -m * a


_ZBLK = 4000
_ZGRID = NE // _ZBLK


def _z_body(nk, final, *refs):
    scal, xe0 = refs[0], refs[1]
    ts = refs[2:2 + nk]
    if final:
        p, qs, p2, q2s, zout, xeout = refs[2 + nk:]
    else:
        p, qs, zout = refs[2 + nk:]
    acc = jnp.dot(xe0[...], p[...], preferred_element_type=jnp.float32)
    if final:
        acc2 = jnp.dot(xe0[...], p2[...], preferred_element_type=jnp.float32)
    for k in range(nk):
        a = scal[k, 0]
        b = scal[k, 1]
        r = jnp.maximum(ts[k][...] * a + b, 0.0)
        acc = acc + jnp.dot(r, qs[k], preferred_element_type=jnp.float32)
        if final:
            acc2 = acc2 + jnp.dot(r, q2s[k], preferred_element_type=jnp.float32)
    zout[...] = acc
    if final:
        xeout[...] = acc2


def _tc_zmat(scal, xe0r, ts, p, qs, p2=None, q2s=None):
    nk = len(ts)
    final = p2 is not None
    in_specs = [pl.BlockSpec(memory_space=pltpu.SMEM),
                pl.BlockSpec((_ZBLK, 16), lambda i: (i, 0))]
    in_specs += [pl.BlockSpec((_ZBLK, 32), lambda i: (i, 0))] * nk
    in_specs += [pl.BlockSpec((16, 64), lambda i: (0, 0)),
                 pl.BlockSpec((nk, 32, 64), lambda i: (0, 0, 0))]
    args = [scal, xe0r] + list(ts) + [p, qs]
    out_specs = [pl.BlockSpec((_ZBLK, 64), lambda i: (i, 0))]
    out_shape = [jax.ShapeDtypeStruct((NE, 64), jnp.float32)]
    if final:
        in_specs += [pl.BlockSpec((16, 16), lambda i: (0, 0)),
                     pl.BlockSpec((nk, 32, 16), lambda i: (0, 0, 0))]
        args += [p2, q2s]
        out_specs += [pl.BlockSpec((_ZBLK, 16), lambda i: (i, 0))]
        out_shape += [jax.ShapeDtypeStruct((NE, 16), jnp.float32)]
    out = pl.pallas_call(
        functools.partial(_z_body, nk, final),
        grid=(_ZGRID,),
        in_specs=in_specs,
        out_specs=tuple(out_specs) if final else out_specs[0],
        out_shape=tuple(out_shape) if final else out_shape[0],
    )(*args)
    return out if final else (out, None)


def _node_body(final, accp, xn, kn2T, wa, wb, *outs):
    u = accp[0] + accp[1]
    m = jnp.mean(u)
    v = jnp.mean(u * u) - m * m
    r = jnp.maximum((u - m) * lax.rsqrt(v + EPS), 0.0)
    x = xn[...] + HSTEP * jnp.dot(r, kn2T[...], preferred_element_type=jnp.float32)
    if final:
        outs[0][...] = jnp.dot(x, wa[...], preferred_element_type=jnp.float32)
    else:
        outs[0][...] = x
        outs[1][...] = jnp.dot(x, wa[...], preferred_element_type=jnp.float32)
        outs[2][...] = jnp.dot(x, wb[...], preferred_element_type=jnp.float32)


def _tc_node_update(accp, xn, kn2T, wa, wb, final):
    if final:
        out_shape = jax.ShapeDtypeStruct((NN, 128), jnp.float32)
    else:
        out_shape = (jax.ShapeDtypeStruct((NN, 128), jnp.float32),
                     jax.ShapeDtypeStruct((NN, 32), jnp.float32),
                     jax.ShapeDtypeStruct((NN, 32), jnp.float32))
    return pl.pallas_call(
        functools.partial(_node_body, final),
        out_shape=out_shape,
    )(accp, xn, kn2T, wa, wb)


def kernel(xn, xe, edge_index, KNopen, KEopen, KE1, KE2, KN1, KN2,
           KNclose, KEclose):
    f32 = jnp.float32
    iInd = edge_index[0].astype(jnp.int32)
    jInd = edge_index[1].astype(jnp.int32)
    ii2d = iInd.reshape(GCH, 128)
    jj2d = jInd.reshape(GCH, 128)
    sidx2d = jnp.stack([iInd, jInd], axis=1).reshape(SCH, 128)
    zrows = jnp.zeros((NN, 32), f32)

    xn_rows = jnp.transpose(xn[0]).astype(f32)
    xe_rows = jnp.transpose(xe[0]).astype(f32)

    wiT, wjT, VT = [], [], []
    for l in range(NL):
        a, b = KE1[l][:, :128], KE1[l][:, 128:]
        wiT.append(jnp.transpose(a + 0.5 * b))
        wjT.append(jnp.transpose(-a + 0.5 * b))
        na, nb = KN1[l][:, :128], KN1[l][:, 128:]
        VT.append(jnp.concatenate([jnp.transpose(na + 0.5 * nb),
                                   jnp.transpose(-na + 0.5 * nb)], axis=1))
    P = [jnp.transpose(KEopen) @ VT[l] for l in range(NL)]
    Q = [jnp.stack([HSTEP * (jnp.transpose(KE2[k]) @ VT[l])
                    for k in range(l + 1)]) for l in range(NL)]
    P2 = jnp.transpose(KEopen) @ jnp.transpose(KEclose)
    Q2 = jnp.stack([HSTEP * (jnp.transpose(KE2[k]) @ jnp.transpose(KEclose))
                    for k in range(NL)])
    kn2T = [jnp.transpose(KN2[l]) for l in range(NL)]

    xnr, yi, yj = _tc_open(xn_rows, jnp.transpose(KNopen), wiT[0], wjT[0])

    ts, scals = [], []
    xe_out_rows = None
    xn_out_rows = None
    for l in range(NL):
        ti, tj = _sc_gather(yi, yj, ii2d, jj2d)
        t, a, b = _tc_add_stats(ti, tj)
        ts.append(t)
        scals.append(jnp.stack([a, b]))
        scal = jnp.stack(scals)
        final = l == NL - 1
        z, xe_out_rows_maybe = _tc_zmat(
            scal, xe_rows, ts, P[l], Q[l],
            P2 if final else None, Q2 if final else None)
        if final:
            xe_out_rows = xe_out_rows_maybe
        accp = _sc_scatter(z.reshape(2 * NE, 32), sidx2d, zrows)
        if final:
            xn_out_rows = _tc_node_update(
                accp, xnr, kn2T[l], jnp.transpose(KNclose),
                jnp.transpose(KNclose), final=True)
        else:
            xnr, yi, yj = _tc_node_update(
                accp, xnr, kn2T[l], wiT[l + 1], wjT[l + 1], final=False)

    xn_out = jnp.transpose(xn_out_rows)[None]
    xe_out = jnp.transpose(xe_out_rows)[None]
    return (xn_out, xe_out)


# speedup vs baseline: 9.5268x; 2.3077x over previous
import functools

import jax
import jax.numpy as jnp
from jax import lax
from jax.experimental import pallas as pl
from jax.experimental.pallas import tpu as pltpu
from jax.experimental.pallas import tpu_sc as plsc

NN = 10000
NE = 320000
NL = 4
HSTEP = 0.1
EPS = 1e-5
NC, NS = 2, 16
NW = NC * NS
GCH = NE // 128
NPT = NN // NS

_SC_MESH = dict(core_axis_name="c", subcore_axis_name="s",
                num_cores=NC, num_subcores=NS)
_SC_PARAMS = pltpu.CompilerParams(use_tc_tiling_on_sc=False)

_GK = 4
_GG = GCH // _GK
_GBASE = _GG // NW
_GEXTRA = _GG - _GBASE * NW
_GMAXCH = (_GBASE + 1) * _GK


def _sc_gather(yi, yj, ii2d, jj2d):
    @functools.partial(
        pl.kernel,
        out_type=(jax.ShapeDtypeStruct((NE, 32), jnp.float32),
                  jax.ShapeDtypeStruct((NE, 32), jnp.float32)),
        mesh=plsc.VectorSubcoreMesh(**_SC_MESH),
        scratch_types=[
            pltpu.VMEM((_GMAXCH, 128), jnp.int32),
            pltpu.VMEM((_GMAXCH, 128), jnp.int32),
            pltpu.VMEM((2, _GK, 128, 32), jnp.float32),
            pltpu.VMEM((2, _GK, 128, 32), jnp.float32),
            pltpu.SemaphoreType.DMA,
            pltpu.SemaphoreType.DMA,
            pltpu.SemaphoreType.DMA,
            pltpu.SemaphoreType.DMA,
        ],
        compiler_params=_SC_PARAMS,
    )
    def k(yi_h, yj_h, ii_h, jj_h, ti_h, tj_h,
          idxi_v, idxj_v, bi_v, bj_v, sg0, sg1, sw0, sw1):
        w = lax.axis_index("s") * NC + lax.axis_index("c")
        ng = _GBASE + jnp.where(w < _GEXTRA, 1, 0)
        c0 = (_GBASE * w + jnp.minimum(w, _GEXTRA)) * _GK

        nbase = _GBASE * _GK
        pltpu.sync_copy(ii_h.at[pl.ds(c0, nbase)], idxi_v.at[pl.ds(0, nbase)])
        pltpu.sync_copy(jj_h.at[pl.ds(c0, nbase)], idxj_v.at[pl.ds(0, nbase)])

        @pl.when(w < _GEXTRA)
        def _():
            pltpu.sync_copy(ii_h.at[pl.ds(c0 + nbase, _GK)],
                            idxi_v.at[pl.ds(nbase, _GK)])
            pltpu.sync_copy(jj_h.at[pl.ds(c0 + nbase, _GK)],
                            idxj_v.at[pl.ds(nbase, _GK)])

        sg = (sg0, sg1)
        sw = (sw0, sw1)

        def issue_g(g, bank):
            for b in range(_GK):
                ch = g * _GK + b
                pltpu.async_copy(yi_h.at[idxi_v.at[ch]], bi_v.at[bank, b], sg[bank])
                pltpu.async_copy(yj_h.at[idxj_v.at[ch]], bj_v.at[bank, b], sg[bank])

        def wait_g(bank):
            for b in range(_GK):
                pltpu.make_async_copy(yi_h.at[idxi_v.at[0]], bi_v.at[bank, b], sg[bank]).wait()
                pltpu.make_async_copy(yj_h.at[idxj_v.at[0]], bj_v.at[bank, b], sg[bank]).wait()

        def issue_w(g, bank):
            for b in range(_GK):
                row = (c0 + g * _GK + b) * 128
                pltpu.async_copy(bi_v.at[bank, b], ti_h.at[pl.ds(row, 128)], sw[bank])
                pltpu.async_copy(bj_v.at[bank, b], tj_h.at[pl.ds(row, 128)], sw[bank])

        def wait_w(bank):
            for b in range(_GK):
                pltpu.make_async_copy(bi_v.at[bank, b], ti_h.at[pl.ds(0, 128)], sw[bank]).wait()
                pltpu.make_async_copy(bj_v.at[bank, b], tj_h.at[pl.ds(0, 128)], sw[bank]).wait()

        issue_g(0, 0)

        def pair(p, carry):
            g1 = 2 * p + 1
            g2 = 2 * p + 2

            @pl.when(g1 < ng)
            def _():
                @pl.when(p > 0)
                def _():
                    wait_w(1)
                issue_g(g1, 1)

            wait_g(0)
            issue_w(2 * p, 0)

            @pl.when(g1 < ng)
            def _():
                @pl.when(g2 < ng)
                def _():
                    wait_w(0)
                    issue_g(g2, 0)
                wait_g(1)
                issue_w(g1, 1)

            return carry

        lax.fori_loop(0, (_GBASE + 1) // 2, pair, 0)
        wait_w(0)
        wait_w(1)

    return k(yi, yj, ii2d, jj2d)


def _sc_scatter(phases, zeros, width, gk, combine=True):
    gg = GCH // gk
    base = gg // NW
    extra = gg - base * NW
    maxch = (base + 1) * gk
    nout = 1 if combine else len(phases)

    @functools.partial(
        pl.kernel,
        out_type=jax.ShapeDtypeStruct((nout, NC, NN, width), jnp.float32),
        mesh=plsc.VectorSubcoreMesh(**_SC_MESH),
        scratch_types=[
            pltpu.VMEM((maxch, 128), jnp.int32),
            pltpu.VMEM((2, gk, 128, width), jnp.float32),
            pltpu.VMEM_SHARED((NN, width), jnp.float32),
            pltpu.SemaphoreType.DMA,
            pltpu.SemaphoreType.DMA,
            pltpu.SemaphoreType.DMA,
            pltpu.SemaphoreType.DMA,
        ],
        compiler_params=_SC_PARAMS,
    )
    def k(*refs):
        nz = len(phases)
        zs = refs[:nz]
        idxs = refs[nz:2 * nz]
        cols = [c for _, _, c in phases]
        zero_h = refs[2 * nz]
        out_h = refs[2 * nz + 1]
        idx_v, val_v, acc_sh, sv0, sv1, ss0, ss1 = refs[2 * nz + 2:]

        cid = lax.axis_index("c")
        sid = lax.axis_index("s")
        w = sid * NC + cid
        sl = pl.ds(sid * NPT, NPT)
        pltpu.sync_copy(zero_h.at[sl], acc_sh.at[sl])
        ng = base + jnp.where(w < extra, 1, 0)
        c0 = (base * w + jnp.minimum(w, extra)) * gk

        nbase = base * gk
        sv = (sv0, sv1)
        ss = (ss0, ss1)
        plsc.subcore_barrier()

        for ph, (z_h, i_h, col) in enumerate(zip(zs, idxs, cols)):
            if not combine and ph > 0:
                pltpu.sync_copy(zero_h.at[sl], acc_sh.at[sl])
                plsc.subcore_barrier()

            pltpu.sync_copy(i_h.at[pl.ds(c0, nbase)], idx_v.at[pl.ds(0, nbase)])

            @pl.when(w < extra)
            def _():
                pltpu.sync_copy(i_h.at[pl.ds(c0 + nbase, gk)],
                                idx_v.at[pl.ds(nbase, gk)])

            def issue_v(g, bank):
                for b in range(gk):
                    row = (c0 + g * gk + b) * 128
                    pltpu.async_copy(z_h.at[pl.ds(row, 128), pl.ds(col, width)],
                                     val_v.at[bank, b], sv[bank])

            def wait_v(bank):
                for b in range(gk):
                    pltpu.make_async_copy(z_h.at[pl.ds(0, 128), pl.ds(col, width)],
                                          val_v.at[bank, b], sv[bank]).wait()

            def issue_s(g, bank):
                for b in range(gk):
                    ch = g * gk + b
                    pltpu.async_copy(val_v.at[bank, b], acc_sh.at[idx_v.at[ch]],
                                     ss[bank], add=True)

            def wait_s(bank):
                for b in range(gk):
                    pltpu.make_async_copy(val_v.at[bank, b], acc_sh.at[idx_v.at[0]],
                                          ss[bank]).wait()

            issue_v(0, 0)

            def pair(p, carry):
                g1 = 2 * p + 1
                g2 = 2 * p + 2

                @pl.when(g1 < ng)
                def _():
                    @pl.when(p > 0)
                    def _():
                        wait_s(1)
                    issue_v(g1, 1)

                wait_v(0)
                issue_s(2 * p, 0)

                @pl.when(g1 < ng)
                def _():
                    @pl.when(g2 < ng)
                    def _():
                        wait_s(0)
                        issue_v(g2, 0)
                    wait_v(1)
                    issue_s(g1, 1)

                return carry

            lax.fori_loop(0, (base + 1) // 2, pair, 0)
            wait_s(0)
            wait_s(1)

            if not combine:
                plsc.subcore_barrier()
                pltpu.sync_copy(acc_sh.at[sl], out_h.at[ph, cid, sl])

        if combine:
            plsc.subcore_barrier()
            pltpu.sync_copy(acc_sh.at[sl], out_h.at[0, cid, sl])

    args = [z for z, _, _ in phases] + [i for _, i, _ in phases] + [zeros]
    return k(*args)


def _kron4(m):
    a, b = m.shape
    return jnp.einsum('ij,ab->iajb', jnp.eye(4, dtype=m.dtype), m).reshape(4 * a, 4 * b)


def _open_node_body(xn4, wo, wi, wj, xo4, yi4, yj4):
    x = jnp.dot(xn4[...], wo[...], preferred_element_type=jnp.float32)
    xo4[...] = x
    yi4[...] = jnp.dot(x, wi[...], preferred_element_type=jnp.float32)
    yj4[...] = jnp.dot(x, wj[...], preferred_element_type=jnp.float32)


def _tc_open_node(xn4, wo4, wi4, wj4):
    return pl.pallas_call(
        _open_node_body,
        out_shape=(jax.ShapeDtypeStruct((NN // 4, 512), jnp.float32),
                   jax.ShapeDtypeStruct((NN // 4, 128), jnp.float32),
                   jax.ShapeDtypeStruct((NN // 4, 128), jnp.float32)),
    )(xn4, wo4, wi4, wj4)


def _open_edge_body(xe_cm, wo, out4):
    out4[...] = lax.dot_general(xe_cm[...], wo[...], (((0,), (0,)), ((), ())),
                                preferred_element_type=jnp.float32)


def _tc_open_edge(xe_cm, wopenT):
    blk = 6400
    return pl.pallas_call(
        _open_edge_body,
        grid=(NE // blk,),
        in_specs=[pl.BlockSpec((16, blk), lambda i: (0, i)),
                  pl.BlockSpec((16, 128), lambda i: (0, 0))],
        out_specs=pl.BlockSpec((blk, 128), lambda i: (i, 0)),
        out_shape=jax.ShapeDtypeStruct((NE, 128), jnp.float32),
    )(xe_cm, wopenT)


_SBLK = 2000
_SGRID = (NE // 4) // _SBLK


def _stats_body(ti4, tj4, t4, s, q):
    x = ti4[...] + tj4[...]
    t4[...] = x
    s[...] = jnp.sum(x, axis=0, keepdims=True)[None]
    q[...] = jnp.sum(x * x, axis=0, keepdims=True)[None]


def _tc_add_stats(ti4, tj4):
    t4, s, q = pl.pallas_call(
        _stats_body,
        grid=(_SGRID,),
        in_specs=[pl.BlockSpec((_SBLK, 128), lambda i: (i, 0)),
                  pl.BlockSpec((_SBLK, 128), lambda i: (i, 0))],
        out_specs=(pl.BlockSpec((_SBLK, 128), lambda i: (i, 0)),
                   pl.BlockSpec((1, 1, 128), lambda i: (i, 0, 0)),
                   pl.BlockSpec((1, 1, 128), lambda i: (i, 0, 0))),
        out_shape=(jax.ShapeDtypeStruct((NE // 4, 128), jnp.float32),
                   jax.ShapeDtypeStruct((_SGRID, 1, 128), jnp.float32),
                   jax.ShapeDtypeStruct((_SGRID, 1, 128), jnp.float32)),
    )(ti4, tj4)
    m = jnp.sum(s) / (NE * 32)
    v = jnp.sum(q) / (NE * 32) - m * m
    a = lax.rsqrt(v + EPS)
    return t4, a, -m * a


_ZBLK = 1000
_ZGRID = (NE // 4) // _ZBLK


def _z_body(nk, final, *refs):
    scal = refs[0]
    ts = refs[1:1 + nk]
    if final:
        qi, qj, q2, zi, zj, xeo = refs[1 + nk:]
    else:
        qi, qj, zi, zj = refs[1 + nk:]
    acc_i = None
    for k in range(nk):
        r = jnp.maximum(ts[k][...] * scal[k, 0] + scal[k, 1], 0.0)
        zik = jnp.dot(r, qi[k], preferred_element_type=jnp.float32)
        zjk = jnp.dot(r, qj[k], preferred_element_type=jnp.float32)
        if acc_i is None:
            acc_i, acc_j = zik, zjk
            if final:
                acc_o = jnp.dot(r, q2[k], preferred_element_type=jnp.float32)
        else:
            acc_i = acc_i + zik
            acc_j = acc_j + zjk
            if final:
                acc_o = acc_o + jnp.dot(r, q2[k], preferred_element_type=jnp.float32)
    zi[...] = acc_i
    zj[...] = acc_j
    if final:
        xeo[...] = acc_o


def _tc_zmat(scal, ts, qi, qj, q2=None):
    nk = len(ts)
    final = q2 is not None
    in_specs = [pl.BlockSpec(memory_space=pltpu.SMEM)]
    in_specs += [pl.BlockSpec((_ZBLK, 128), lambda i: (i, 0))] * nk
    in_specs += [pl.BlockSpec((nk, 128, 128), lambda i: (0, 0, 0))] * 2
    args = [scal] + list(ts) + [qi, qj]
    out_specs = [pl.BlockSpec((_ZBLK, 128), lambda i: (i, 0))] * 2
    out_shape = [jax.ShapeDtypeStruct((NE // 4, 128), jnp.float32)] * 2
    if final:
        in_specs += [pl.BlockSpec((nk, 128, 64), lambda i: (0, 0, 0))]
        args += [q2]
        out_specs += [pl.BlockSpec((_ZBLK, 64), lambda i: (i, 0))]
        out_shape += [jax.ShapeDtypeStruct((NE // 4, 64), jnp.float32)]
    return pl.pallas_call(
        functools.partial(_z_body, nk, final),
        grid=(_ZGRID,),
        in_specs=in_specs,
        out_specs=tuple(out_specs),
        out_shape=tuple(out_shape),
    )(*args)


def _node_body(final, accp, slo, shi, xn4, vilo, vihi, vjlo, vjhi,
               kn2, wa, wb, *outs):
    u = accp[0] + accp[1]
    u = u + jnp.dot(slo[0, 0] + slo[0, 1], vilo[...], preferred_element_type=jnp.float32)
    u = u + jnp.dot(shi[0, 0] + shi[0, 1], vihi[...], preferred_element_type=jnp.float32)
    u = u + jnp.dot(slo[1, 0] + slo[1, 1], vjlo[...], preferred_element_type=jnp.float32)
    u = u + jnp.dot(shi[1, 0] + shi[1, 1], vjhi[...], preferred_element_type=jnp.float32)
    m = jnp.mean(u)
    v = jnp.mean(u * u) - m * m
    r = jnp.maximum((u - m) * lax.rsqrt(v + EPS), 0.0)
    x = xn4[...] + HSTEP * jnp.dot(r, kn2[...], preferred_element_type=jnp.float32)
    if final:
        outs[0][...] = jnp.dot(x, wa[...], preferred_element_type=jnp.float32)
    else:
        outs[0][...] = x
        outs[1][...] = jnp.dot(x, wa[...], preferred_element_type=jnp.float32)
        outs[2][...] = jnp.dot(x, wb[...], preferred_element_type=jnp.float32)


def _tc_node_update(accp4, slo4, shi4, xn4, vk, kn2k, wa, wb, final):
    if final:
        out_shape = jax.ShapeDtypeStruct((NN // 4, 512), jnp.float32)
    else:
        out_shape = (jax.ShapeDtypeStruct((NN // 4, 512), jnp.float32),
                     jax.ShapeDtypeStruct((NN // 4, 128), jnp.float32),
                     jax.ShapeDtypeStruct((NN // 4, 128), jnp.float32))
    return pl.pallas_call(
        functools.partial(_node_body, final),
        out_shape=out_shape,
    )(accp4, slo4, shi4, xn4, *vk, kn2k, wa, wb)


def _xeout_body(xe_cm, reluT, w2, out):
    out[...] = (jnp.dot(w2[...], xe_cm[...], preferred_element_type=jnp.float32)
                + reluT[...])


def _tc_xe_out(xe_cm, reluT_cm, w2):
    blk = 6400
    return pl.pallas_call(
        _xeout_body,
        grid=(NE // blk,),
        in_specs=[pl.BlockSpec((16, blk), lambda i: (0, i)),
                  pl.BlockSpec((16, blk), lambda i: (0, i)),
                  pl.BlockSpec((16, 16), lambda i: (0, 0))],
        out_specs=pl.BlockSpec((16, blk), lambda i: (0, i)),
        out_shape=jax.ShapeDtypeStruct((16, NE), jnp.float32),
    )(xe_cm, reluT_cm, w2)


def kernel(xn, xe, edge_index, KNopen, KEopen, KE1, KE2, KN1, KN2,
           KNclose, KEclose):
    f32 = jnp.float32
    iInd = edge_index[0].astype(jnp.int32)
    jInd = edge_index[1].astype(jnp.int32)
    ii2d = iInd.reshape(GCH, 128)
    jj2d = jInd.reshape(GCH, 128)
    zeros32 = jnp.zeros((NN // 4, 128), f32).reshape(NN, 32)
    zeros64 = jnp.zeros((NN // 2, 128), f32).reshape(NN, 64)

    xn4 = jnp.transpose(xn[0]).reshape(NN // 4, 512)
    xe_cm = xe[0]

    wiT, wjT, ViT, VjT = [], [], [], []
    for l in range(NL):
        a, b = KE1[l][:, :128], KE1[l][:, 128:]
        wiT.append(jnp.transpose(a + 0.5 * b))
        wjT.append(jnp.transpose(-a + 0.5 * b))
        na, nb = KN1[l][:, :128], KN1[l][:, 128:]
        ViT.append(jnp.transpose(na + 0.5 * nb))
        VjT.append(jnp.transpose(-na + 0.5 * nb))
    QiK = [jnp.stack([_kron4(HSTEP * (jnp.transpose(KE2[k]) @ ViT[l]))
                      for k in range(l + 1)]) for l in range(NL)]
    QjK = [jnp.stack([_kron4(HSTEP * (jnp.transpose(KE2[k]) @ VjT[l]))
                      for k in range(l + 1)]) for l in range(NL)]
    Q2K = jnp.stack([_kron4(HSTEP * (jnp.transpose(KE2[k]) @ jnp.transpose(KEclose)))
                     for k in range(NL)])
    kn2K = [_kron4(jnp.transpose(KN2[l])) for l in range(NL)]
    wiK = [_kron4(w) for w in wiT]
    wjK = [_kron4(w) for w in wjT]
    VK = [(_kron4(ViT[l][:64]), _kron4(ViT[l][64:]),
           _kron4(VjT[l][:64]), _kron4(VjT[l][64:])) for l in range(NL)]
    wopenK = _kron4(jnp.transpose(KNopen))
    wcloseK = _kron4(jnp.transpose(KNclose))
    w2 = KEclose @ KEopen

    xn4, yi4, yj4 = _tc_open_node(xn4, wopenK, wiK[0], wjK[0])
    xe0r = _tc_open_edge(xe_cm, jnp.transpose(KEopen))

    slo = _sc_scatter([(xe0r, ii2d, 0), (xe0r, jj2d, 0)],
                      zeros64, 64, 4, combine=False)
    shi = _sc_scatter([(xe0r, ii2d, 64), (xe0r, jj2d, 64)],
                      zeros64, 64, 4, combine=False)
    slo4 = slo.reshape(2, NC, NN // 4, 256)
    shi4 = shi.reshape(2, NC, NN // 4, 256)

    ts, scals = [], []
    xeo4 = None
    xn_out4 = None
    for l in range(NL):
        ti, tj = _sc_gather(yi4.reshape(NN, 32), yj4.reshape(NN, 32), ii2d, jj2d)
        t4, a, b = _tc_add_stats(ti.reshape(NE // 4, 128), tj.reshape(NE // 4, 128))
        ts.append(t4)
        scals.append(jnp.stack([a, b]))
        scal = jnp.stack(scals)
        final = l == NL - 1
        if final:
            zi4, zj4, xeo4 = _tc_zmat(scal, ts, QiK[l], QjK[l], Q2K)
        else:
            zi4, zj4 = _tc_zmat(scal, ts, QiK[l], QjK[l])
        accp = _sc_scatter([(zi4.reshape(NE, 32), ii2d, 0),
                            (zj4.reshape(NE, 32), jj2d, 0)], zeros32, 32, 4)
        accp4 = accp.reshape(NC, NN // 4, 128)
        if final:
            xn_out4 = _tc_node_update(accp4, slo4, shi4, xn4, VK[l],
                                      kn2K[l], wcloseK, wcloseK, final=True)
        else:
            xn4, yi4, yj4 = _tc_node_update(accp4, slo4, shi4, xn4, VK[l],
                                            kn2K[l], wiK[l + 1], wjK[l + 1],
                                            final=False)

    xn_out = jnp.transpose(xn_out4.reshape(NN, 128))[None]
    reluT = jnp.transpose(xeo4.reshape(NE, 16))
    xe_out = _tc_xe_out(xe_cm, reluT, w2)[None]
    return (xn_out, xe_out)


# speedup vs baseline: 10.1702x; 1.0675x over previous
import functools

import jax
import jax.numpy as jnp
from jax import lax
from jax.experimental import pallas as pl
from jax.experimental.pallas import tpu as pltpu
from jax.experimental.pallas import tpu_sc as plsc

NN = 10000
NE = 320000
NL = 4
HSTEP = 0.1
EPS = 1e-5
NC, NS = 2, 16
NW = NC * NS
GCH = NE // 128
NPT = NN // NS

_SC_MESH = dict(core_axis_name="c", subcore_axis_name="s",
                num_cores=NC, num_subcores=NS)
_SC_PARAMS = pltpu.CompilerParams(use_tc_tiling_on_sc=False)

_GK = 4
_GG = GCH // _GK
_GBASE = _GG // NW
_GEXTRA = _GG - _GBASE * NW
_GMAXCH = (_GBASE + 1) * _GK


def _sc_gather(yi, yj, ii2d, jj2d):
    @functools.partial(
        pl.kernel,
        out_type=jax.ShapeDtypeStruct((NE, 32), jnp.float32),
        mesh=plsc.VectorSubcoreMesh(**_SC_MESH),
        scratch_types=[
            pltpu.VMEM((_GMAXCH, 128), jnp.int32),
            pltpu.VMEM((_GMAXCH, 128), jnp.int32),
            pltpu.VMEM((2, _GK, 128, 32), jnp.float32),
            pltpu.VMEM((2, _GK, 128, 32), jnp.float32),
            pltpu.SemaphoreType.DMA,
            pltpu.SemaphoreType.DMA,
            pltpu.SemaphoreType.DMA,
            pltpu.SemaphoreType.DMA,
        ],
        compiler_params=_SC_PARAMS,
    )
    def k(yi_h, yj_h, ii_h, jj_h, t_h,
          idxi_v, idxj_v, bi_v, bj_v, sg0, sg1, sw0, sw1):
        w = lax.axis_index("s") * NC + lax.axis_index("c")
        ng = _GBASE + jnp.where(w < _GEXTRA, 1, 0)
        c0 = (_GBASE * w + jnp.minimum(w, _GEXTRA)) * _GK

        nbase = _GBASE * _GK
        pltpu.sync_copy(ii_h.at[pl.ds(c0, nbase)], idxi_v.at[pl.ds(0, nbase)])
        pltpu.sync_copy(jj_h.at[pl.ds(c0, nbase)], idxj_v.at[pl.ds(0, nbase)])

        @pl.when(w < _GEXTRA)
        def _():
            pltpu.sync_copy(ii_h.at[pl.ds(c0 + nbase, _GK)],
                            idxi_v.at[pl.ds(nbase, _GK)])
            pltpu.sync_copy(jj_h.at[pl.ds(c0 + nbase, _GK)],
                            idxj_v.at[pl.ds(nbase, _GK)])

        sg = (sg0, sg1)
        sw = (sw0, sw1)

        def issue_g(g, bank):
            for b in range(_GK):
                ch = g * _GK + b
                pltpu.async_copy(yi_h.at[idxi_v.at[ch]], bi_v.at[bank, b], sg[bank])
                pltpu.async_copy(yj_h.at[idxj_v.at[ch]], bj_v.at[bank, b], sg[bank])

        def wait_g(bank):
            for b in range(_GK):
                pltpu.make_async_copy(yi_h.at[idxi_v.at[0]], bi_v.at[bank, b], sg[bank]).wait()
                pltpu.make_async_copy(yj_h.at[idxj_v.at[0]], bj_v.at[bank, b], sg[bank]).wait()

        def add_w(g, bank):
            for b in range(_GK):
                bi = bi_v.at[bank, b]
                bj = bj_v.at[bank, b]

                def addrow(r, carry):
                    for c in (0, 16):
                        bi[r, pl.ds(c, 16)] = bi[r, pl.ds(c, 16)] + bj[r, pl.ds(c, 16)]
                    return carry

                lax.fori_loop(0, 128, addrow, 0)
                row = (c0 + g * _GK + b) * 128
                pltpu.async_copy(bi, t_h.at[pl.ds(row, 128)], sw[bank])

        def wait_w(bank):
            for b in range(_GK):
                pltpu.make_async_copy(bi_v.at[bank, b], t_h.at[pl.ds(0, 128)], sw[bank]).wait()

        issue_g(0, 0)

        def pair(p, carry):
            g1 = 2 * p + 1
            g2 = 2 * p + 2

            @pl.when(g1 < ng)
            def _():
                @pl.when(p > 0)
                def _():
                    wait_w(1)
                issue_g(g1, 1)

            wait_g(0)
            add_w(2 * p, 0)

            @pl.when(g1 < ng)
            def _():
                @pl.when(g2 < ng)
                def _():
                    wait_w(0)
                    issue_g(g2, 0)
                wait_g(1)
                add_w(g1, 1)

            return carry

        lax.fori_loop(0, (_GBASE + 1) // 2, pair, 0)
        wait_w(0)
        wait_w(1)

    return k(yi, yj, ii2d, jj2d)


def _sc_scatter(phases, zeros, width, gk, combine=True):
    gg = GCH // gk
    base = gg // NW
    extra = gg - base * NW
    maxch = (base + 1) * gk
    nout = 1 if combine else len(phases)

    @functools.partial(
        pl.kernel,
        out_type=jax.ShapeDtypeStruct((nout, NC, NN, width), jnp.float32),
        mesh=plsc.VectorSubcoreMesh(**_SC_MESH),
        scratch_types=[
            pltpu.VMEM((maxch, 128), jnp.int32),
            pltpu.VMEM((2, gk, 128, width), jnp.float32),
            pltpu.VMEM_SHARED((NN, width), jnp.float32),
            pltpu.SemaphoreType.DMA,
            pltpu.SemaphoreType.DMA,
            pltpu.SemaphoreType.DMA,
            pltpu.SemaphoreType.DMA,
        ],
        compiler_params=_SC_PARAMS,
    )
    def k(*refs):
        nz = len(phases)
        zs = refs[:nz]
        idxs = refs[nz:2 * nz]
        cols = [c for _, _, c in phases]
        zero_h = refs[2 * nz]
        out_h = refs[2 * nz + 1]
        idx_v, val_v, acc_sh, sv0, sv1, ss0, ss1 = refs[2 * nz + 2:]

        cid = lax.axis_index("c")
        sid = lax.axis_index("s")
        w = sid * NC + cid
        sl = pl.ds(sid * NPT, NPT)
        pltpu.sync_copy(zero_h.at[sl], acc_sh.at[sl])
        ng = base + jnp.where(w < extra, 1, 0)
        c0 = (base * w + jnp.minimum(w, extra)) * gk

        nbase = base * gk
        sv = (sv0, sv1)
        ss = (ss0, ss1)
        plsc.subcore_barrier()

        for ph, (z_h, i_h, col) in enumerate(zip(zs, idxs, cols)):
            if not combine and ph > 0:
                pltpu.sync_copy(zero_h.at[sl], acc_sh.at[sl])
                plsc.subcore_barrier()

            pltpu.sync_copy(i_h.at[pl.ds(c0, nbase)], idx_v.at[pl.ds(0, nbase)])

            @pl.when(w < extra)
            def _():
                pltpu.sync_copy(i_h.at[pl.ds(c0 + nbase, gk)],
                                idx_v.at[pl.ds(nbase, gk)])

            def issue_v(g, bank):
                for b in range(gk):
                    row = (c0 + g * gk + b) * 128
                    pltpu.async_copy(z_h.at[pl.ds(row, 128), pl.ds(col, width)],
                                     val_v.at[bank, b], sv[bank])

            def wait_v(bank):
                for b in range(gk):
                    pltpu.make_async_copy(z_h.at[pl.ds(0, 128), pl.ds(col, width)],
                                          val_v.at[bank, b], sv[bank]).wait()

            def issue_s(g, bank):
                for b in range(gk):
                    ch = g * gk + b
                    pltpu.async_copy(val_v.at[bank, b], acc_sh.at[idx_v.at[ch]],
                                     ss[bank], add=True)

            def wait_s(bank):
                for b in range(gk):
                    pltpu.make_async_copy(val_v.at[bank, b], acc_sh.at[idx_v.at[0]],
                                          ss[bank]).wait()

            issue_v(0, 0)

            def pair(p, carry):
                g1 = 2 * p + 1
                g2 = 2 * p + 2

                @pl.when(g1 < ng)
                def _():
                    @pl.when(p > 0)
                    def _():
                        wait_s(1)
                    issue_v(g1, 1)

                wait_v(0)
                issue_s(2 * p, 0)

                @pl.when(g1 < ng)
                def _():
                    @pl.when(g2 < ng)
                    def _():
                        wait_s(0)
                        issue_v(g2, 0)
                    wait_v(1)
                    issue_s(g1, 1)

                return carry

            lax.fori_loop(0, (base + 1) // 2, pair, 0)
            wait_s(0)
            wait_s(1)

            if not combine:
                plsc.subcore_barrier()
                pltpu.sync_copy(acc_sh.at[sl], out_h.at[ph, cid, sl])

        if combine:
            plsc.subcore_barrier()
            pltpu.sync_copy(acc_sh.at[sl], out_h.at[0, cid, sl])

    args = [z for z, _, _ in phases] + [i for _, i, _ in phases] + [zeros]
    return k(*args)


def _kron4(m):
    a, b = m.shape
    return jnp.einsum('ij,ab->iajb', jnp.eye(4, dtype=m.dtype), m).reshape(4 * a, 4 * b)


def _open_node_body(xn4, wo, wi, wj, xo4, yi4, yj4):
    x = jnp.dot(xn4[...], wo[...], preferred_element_type=jnp.float32)
    xo4[...] = x
    yi4[...] = jnp.dot(x, wi[...], preferred_element_type=jnp.float32)
    yj4[...] = jnp.dot(x, wj[...], preferred_element_type=jnp.float32)


def _tc_open_node(xn4, wo4, wi4, wj4):
    return pl.pallas_call(
        _open_node_body,
        out_shape=(jax.ShapeDtypeStruct((NN // 4, 512), jnp.float32),
                   jax.ShapeDtypeStruct((NN // 4, 128), jnp.float32),
                   jax.ShapeDtypeStruct((NN // 4, 128), jnp.float32)),
    )(xn4, wo4, wi4, wj4)


def _open_edge_body(xe_cm, wo, out4):
    out4[...] = lax.dot_general(xe_cm[...], wo[...], (((0,), (0,)), ((), ())),
                                preferred_element_type=jnp.float32)


def _tc_open_edge(xe_cm, wopenT):
    blk = 6400
    return pl.pallas_call(
        _open_edge_body,
        grid=(NE // blk,),
        in_specs=[pl.BlockSpec((16, blk), lambda i: (0, i)),
                  pl.BlockSpec((16, 128), lambda i: (0, 0))],
        out_specs=pl.BlockSpec((blk, 128), lambda i: (i, 0)),
        out_shape=jax.ShapeDtypeStruct((NE, 128), jnp.float32),
    )(xe_cm, wopenT)


_SBLK = 2000
_SGRID = (NE // 4) // _SBLK


def _stats_body(t4, s, q):
    x = t4[...]
    s[...] = jnp.sum(x, axis=0, keepdims=True)[None]
    q[...] = jnp.sum(x * x, axis=0, keepdims=True)[None]


def _tc_add_stats(t4):
    s, q = pl.pallas_call(
        _stats_body,
        grid=(_SGRID,),
        in_specs=[pl.BlockSpec((_SBLK, 128), lambda i: (i, 0))],
        out_specs=(pl.BlockSpec((1, 1, 128), lambda i: (i, 0, 0)),
                   pl.BlockSpec((1, 1, 128), lambda i: (i, 0, 0))),
        out_shape=(jax.ShapeDtypeStruct((_SGRID, 1, 128), jnp.float32),
                   jax.ShapeDtypeStruct((_SGRID, 1, 128), jnp.float32)),
    )(t4)
    m = jnp.sum(s) / (NE * 32)
    v = jnp.sum(q) / (NE * 32) - m * m
    a = lax.rsqrt(v + EPS)
    return a, -m * a


_ZBLK = 1000
_ZGRID = (NE // 4) // _ZBLK


def _z_body(nk, final, *refs):
    scal = refs[0]
    ts = refs[1:1 + nk]
    if final:
        qi, qj, q2, zi, zj, xeo = refs[1 + nk:]
    else:
        qi, qj, zi, zj = refs[1 + nk:]
    acc_i = None
    for k in range(nk):
        r = jnp.maximum(ts[k][...] * scal[k, 0] + scal[k, 1], 0.0)
        zik = jnp.dot(r, qi[k], preferred_element_type=jnp.float32)
        zjk = jnp.dot(r, qj[k], preferred_element_type=jnp.float32)
        if acc_i is None:
            acc_i, acc_j = zik, zjk
            if final:
                acc_o = jnp.dot(r, q2[k], preferred_element_type=jnp.float32)
        else:
            acc_i = acc_i + zik
            acc_j = acc_j + zjk
            if final:
                acc_o = acc_o + jnp.dot(r, q2[k], preferred_element_type=jnp.float32)
    zi[...] = acc_i
    zj[...] = acc_j
    if final:
        xeo[...] = acc_o


def _tc_zmat(scal, ts, qi, qj, q2=None):
    nk = len(ts)
    final = q2 is not None
    in_specs = [pl.BlockSpec(memory_space=pltpu.SMEM)]
    in_specs += [pl.BlockSpec((_ZBLK, 128), lambda i: (i, 0))] * nk
    in_specs += [pl.BlockSpec((nk, 128, 128), lambda i: (0, 0, 0))] * 2
    args = [scal] + list(ts) + [qi, qj]
    out_specs = [pl.BlockSpec((_ZBLK, 128), lambda i: (i, 0))] * 2
    out_shape = [jax.ShapeDtypeStruct((NE // 4, 128), jnp.float32)] * 2
    if final:
        in_specs += [pl.BlockSpec((nk, 128, 64), lambda i: (0, 0, 0))]
        args += [q2]
        out_specs += [pl.BlockSpec((_ZBLK, 64), lambda i: (i, 0))]
        out_shape += [jax.ShapeDtypeStruct((NE // 4, 64), jnp.float32)]
    return pl.pallas_call(
        functools.partial(_z_body, nk, final),
        grid=(_ZGRID,),
        in_specs=in_specs,
        out_specs=tuple(out_specs),
        out_shape=tuple(out_shape),
    )(*args)


def _node_body(final, accp, slo, shi, xn4, vilo, vihi, vjlo, vjhi,
               kn2, wa, wb, *outs):
    u = accp[0] + accp[1]
    u = u + jnp.dot(slo[0, 0] + slo[0, 1], vilo[...], preferred_element_type=jnp.float32)
    u = u + jnp.dot(shi[0, 0] + shi[0, 1], vihi[...], preferred_element_type=jnp.float32)
    u = u + jnp.dot(slo[1, 0] + slo[1, 1], vjlo[...], preferred_element_type=jnp.float32)
    u = u + jnp.dot(shi[1, 0] + shi[1, 1], vjhi[...], preferred_element_type=jnp.float32)
    m = jnp.mean(u)
    v = jnp.mean(u * u) - m * m
    r = jnp.maximum((u - m) * lax.rsqrt(v + EPS), 0.0)
    x = xn4[...] + HSTEP * jnp.dot(r, kn2[...], preferred_element_type=jnp.float32)
    if final:
        outs[0][...] = jnp.dot(x, wa[...], preferred_element_type=jnp.float32)
    else:
        outs[0][...] = x
        outs[1][...] = jnp.dot(x, wa[...], preferred_element_type=jnp.float32)
        outs[2][...] = jnp.dot(x, wb[...], preferred_element_type=jnp.float32)


def _tc_node_update(accp4, slo4, shi4, xn4, vk, kn2k, wa, wb, final):
    if final:
        out_shape = jax.ShapeDtypeStruct((NN // 4, 512), jnp.float32)
    else:
        out_shape = (jax.ShapeDtypeStruct((NN // 4, 512), jnp.float32),
                     jax.ShapeDtypeStruct((NN // 4, 128), jnp.float32),
                     jax.ShapeDtypeStruct((NN // 4, 128), jnp.float32))
    return pl.pallas_call(
        functools.partial(_node_body, final),
        out_shape=out_shape,
    )(accp4, slo4, shi4, xn4, *vk, kn2k, wa, wb)


def _xeout_body(xe_cm, reluT, w2, out):
    out[...] = (jnp.dot(w2[...], xe_cm[...], preferred_element_type=jnp.float32)
                + reluT[...])


def _tc_xe_out(xe_cm, reluT_cm, w2):
    blk = 6400
    return pl.pallas_call(
        _xeout_body,
        grid=(NE // blk,),
        in_specs=[pl.BlockSpec((16, blk), lambda i: (0, i)),
                  pl.BlockSpec((16, blk), lambda i: (0, i)),
                  pl.BlockSpec((16, 16), lambda i: (0, 0))],
        out_specs=pl.BlockSpec((16, blk), lambda i: (0, i)),
        out_shape=jax.ShapeDtypeStruct((16, NE), jnp.float32),
    )(xe_cm, reluT_cm, w2)


def kernel(xn, xe, edge_index, KNopen, KEopen, KE1, KE2, KN1, KN2,
           KNclose, KEclose):
    f32 = jnp.float32
    iInd = edge_index[0].astype(jnp.int32)
    jInd = edge_index[1].astype(jnp.int32)
    ii2d = iInd.reshape(GCH, 128)
    jj2d = jInd.reshape(GCH, 128)
    zeros32 = jnp.zeros((NN // 4, 128), f32).reshape(NN, 32)
    zeros64 = jnp.zeros((NN // 2, 128), f32).reshape(NN, 64)

    xn4 = jnp.transpose(xn[0]).reshape(NN // 4, 512)
    xe_cm = xe[0]

    wiT, wjT, ViT, VjT = [], [], [], []
    for l in range(NL):
        a, b = KE1[l][:, :128], KE1[l][:, 128:]
        wiT.append(jnp.transpose(a + 0.5 * b))
        wjT.append(jnp.transpose(-a + 0.5 * b))
        na, nb = KN1[l][:, :128], KN1[l][:, 128:]
        ViT.append(jnp.transpose(na + 0.5 * nb))
        VjT.append(jnp.transpose(-na + 0.5 * nb))
    QiK = [jnp.stack([_kron4(HSTEP * (jnp.transpose(KE2[k]) @ ViT[l]))
                      for k in range(l + 1)]) for l in range(NL)]
    QjK = [jnp.stack([_kron4(HSTEP * (jnp.transpose(KE2[k]) @ VjT[l]))
                      for k in range(l + 1)]) for l in range(NL)]
    Q2K = jnp.stack([_kron4(HSTEP * (jnp.transpose(KE2[k]) @ jnp.transpose(KEclose)))
                     for k in range(NL)])
    kn2K = [_kron4(jnp.transpose(KN2[l])) for l in range(NL)]
    wiK = [_kron4(w) for w in wiT]
    wjK = [_kron4(w) for w in wjT]
    VK = [(_kron4(ViT[l][:64]), _kron4(ViT[l][64:]),
           _kron4(VjT[l][:64]), _kron4(VjT[l][64:])) for l in range(NL)]
    wopenK = _kron4(jnp.transpose(KNopen))
    wcloseK = _kron4(jnp.transpose(KNclose))
    w2 = KEclose @ KEopen

    xn4, yi4, yj4 = _tc_open_node(xn4, wopenK, wiK[0], wjK[0])
    xe0r = _tc_open_edge(xe_cm, jnp.transpose(KEopen))

    slo = _sc_scatter([(xe0r, ii2d, 0), (xe0r, jj2d, 0)],
                      zeros64, 64, 4, combine=False)
    shi = _sc_scatter([(xe0r, ii2d, 64), (xe0r, jj2d, 64)],
                      zeros64, 64, 4, combine=False)
    slo4 = slo.reshape(2, NC, NN // 4, 256)
    shi4 = shi.reshape(2, NC, NN // 4, 256)

    ts, scals = [], []
    xeo4 = None
    xn_out4 = None
    for l in range(NL):
        t = _sc_gather(yi4.reshape(NN, 32), yj4.reshape(NN, 32), ii2d, jj2d)
        t4 = t.reshape(NE // 4, 128)
        a, b = _tc_add_stats(t4)
        ts.append(t4)
        scals.append(jnp.stack([a, b]))
        scal = jnp.stack(scals)
        final = l == NL - 1
        if final:
            zi4, zj4, xeo4 = _tc_zmat(scal, ts, QiK[l], QjK[l], Q2K)
        else:
            zi4, zj4 = _tc_zmat(scal, ts, QiK[l], QjK[l])
        accp = _sc_scatter([(zi4.reshape(NE, 32), ii2d, 0),
                            (zj4.reshape(NE, 32), jj2d, 0)], zeros32, 32, 4)
        accp4 = accp.reshape(NC, NN // 4, 128)
        if final:
            xn_out4 = _tc_node_update(accp4, slo4, shi4, xn4, VK[l],
                                      kn2K[l], wcloseK, wcloseK, final=True)
        else:
            xn4, yi4, yj4 = _tc_node_update(accp4, slo4, shi4, xn4, VK[l],
                                            kn2K[l], wiK[l + 1], wjK[l + 1],
                                            final=False)

    xn_out = jnp.transpose(xn_out4.reshape(NN, 128))[None]
    reluT = jnp.transpose(xeo4.reshape(NE, 16))
    xe_out = _tc_xe_out(xe_cm, reluT, w2)[None]
    return (xn_out, xe_out)
